# baseline (speedup 1.0000x reference)
import functools

import numpy as np
import jax
import jax.numpy as jnp
from jax import lax
from jax.experimental import pallas as pl
from jax.experimental.pallas import tpu as pltpu
from jax.experimental.pallas import tpu_sc as plsc

_B, _N, _T = 1024, 100, 200
_NROWS = _B * _N
_D = _T
_L = 16
_NCH = 13
_OFFS = tuple(min(_L * c, _D - _L) for c in range(_NCH))
_NC, _NS = 2, 16
_NW = _NC * _NS
_BPW = _B // _NW
_IDXSTRIDE = 256

_DOPING_RATIO = 0.15

_ARG_SHAPES = (
    ((_B, _N, _T), jnp.float32),
    ((_NROWS * _L,), jnp.int32),
    ((_B * _IDXSTRIDE,), jnp.int32),
)


@functools.cache
def _build_consts():
    with jax.ensure_compile_time_eval():
        mask = np.asarray(
            jax.random.bernoulli(jax.random.key(42), _DOPING_RATIO, (_NROWS, _D)))
        perm = np.asarray(
            jax.random.permutation(jax.random.key(43), _NROWS)).astype(np.int32)
    cols = np.asarray(_OFFS)[:, None] + np.arange(_L)[None, :]
    bits = mask[:, cols].astype(np.uint32)
    shifts = (31 - np.arange(_L, dtype=np.uint32))[None, None, :]
    words = (bits << shifts).sum(-1, dtype=np.uint32)
    words = np.pad(words, ((0, 0), (0, _L - _NCH)))
    idx = np.zeros((_B, _IDXSTRIDE), np.int32)
    pr = perm.reshape(_B, _N)
    idx[:, :_N] = pr // _N
    idx[:, 128:128 + _N] = pr % _N
    return words.astype(np.int32).reshape(-1), idx.reshape(-1)


def _swap_noise_body(x_hbm, maskw_hbm, idx_hbm, out_hbm,
                     idx_v, mw0, mw1, orig0, orig1, swap0, swap1,
                     isem0, isem1, gsem0, gsem1, osem0, osem1):
    wid = lax.axis_index("s") * _NC + lax.axis_index("c")
    bt0 = wid * _BPW
    pltpu.sync_copy(idx_hbm.at[pl.ds(bt0 * _IDXSTRIDE, _BPW * _IDXSTRIDE)], idx_v)

    iot = lax.iota(jnp.int32, _L)
    zerov = jnp.full((_L,), 0, jnp.int32)
    colv = [iot + off for off in _OFFS]
    mws = (mw0, mw1)
    origs = (orig0, orig1)
    swaps = (swap0, swap1)
    isems = (isem0, isem1)
    gsems = (gsem0, gsem1)
    osems = (osem0, osem1)

    def step(j, carry):
        p = 0
        bt = bt0 + j
        handles = [
            pltpu.async_copy(x_hbm.at[pl.ds(bt, 1)], origs[p], isems[p]),
            pltpu.async_copy(
                maskw_hbm.at[pl.ds(bt * (_N * _L), _N * _L)], mws[p], isems[p]),
        ]
        for g in range(7):
            cnt = _L if g < 6 else _N - 6 * _L
            pbv = idx_v[pl.ds(j * _IDXSTRIDE + g * _L, _L)]
            pnv = idx_v[pl.ds(j * _IDXSTRIDE + 128 + g * _L, _L)]
            for l in range(cnt):
                i = g * _L + l
                handles.append(pltpu.async_copy(
                    x_hbm.at[pl.ds(pbv[l], 1), pl.ds(pnv[l], 1), :],
                    swaps[p].at[pl.ds(i, 1)], gsems[p]))
        for h in handles:
            h.wait()

        mwb, swb, orb = mws[p], swaps[p], origs[p]

        @plsc.parallel_loop(0, _N, 1, unroll=2)
        def _row(r):
            wrow = mwb[pl.ds(r * _L, _L)]
            rv = jnp.full((_L,), r, jnp.int32)
            for c in range(_NCH):
                cv = jnp.full((_L,), c, jnp.int32)
                wb = wrow.at[cv].get(mode="promise_in_bounds")
                msk = lax.shift_left(wb, iot) < 0
                sv = swb[r, 0, pl.ds(_OFFS[c], _L)]
                plsc.store_scatter(orb, [zerov, rv, colv[c]], sv, mask=msk)

        pltpu.async_copy(orb, out_hbm.at[pl.ds(bt, 1)], osems[p]).wait()
        return carry

    lax.fori_loop(0, _BPW, step, 0)


@functools.cache
def _swap_noise():
    mesh = plsc.VectorSubcoreMesh(
        core_axis_name="c", subcore_axis_name="s",
        num_cores=_NC, num_subcores=_NS)
    return pl.kernel(
        _swap_noise_body,
        out_type=jax.ShapeDtypeStruct((_B, _N, _T), jnp.float32),
        mesh=mesh,
        compiler_params=pltpu.CompilerParams(
            use_tc_tiling_on_sc=True, needs_layout_passes=False),
        scratch_types=[
            pltpu.VMEM((_BPW * _IDXSTRIDE,), jnp.int32),
            pltpu.VMEM((_N * _L,), jnp.int32),
            pltpu.VMEM((_N * _L,), jnp.int32),
            pltpu.VMEM((1, _N, _D), jnp.float32),
            pltpu.VMEM((1, _N, _D), jnp.float32),
            pltpu.VMEM((_N, 1, _D), jnp.float32),
            pltpu.VMEM((_N, 1, _D), jnp.float32),
            pltpu.SemaphoreType.DMA,
            pltpu.SemaphoreType.DMA,
            pltpu.SemaphoreType.DMA,
            pltpu.SemaphoreType.DMA,
            pltpu.SemaphoreType.DMA,
            pltpu.SemaphoreType.DMA,
        ],
    )


def kernel(x):
    maskw, idx = _build_consts()
    return _swap_noise()(x, jnp.asarray(maskw), jnp.asarray(idx))
